# baseline (speedup 1.0000x reference)
import jax
import jax.numpy as jnp
from jax import lax
from jax.experimental import pallas as pl
from jax.experimental.pallas import tpu as pltpu, tpu_sc as plsc

B = 16384
F = 26
V = 100000

NC = 2
NS = 16
L = 16
VPAD = 100000
FH = 13
NW = NC * NS
BPW = B // NW
CHUNK = 128
NCHUNK = BPW // CHUNK


def _body(xt_hbm, ta_hbm, tb_hbm, out_hbm, idxv, valsv, outv, sem):
    wid = lax.axis_index("s") * NC + lax.axis_index("c")
    base = wid * BPW

    pltpu.sync_copy(xt_hbm.at[:, wid], idxv)

    def make_off_fire(table_hbm, lo):
        def off_fire_f(f, _):
            fbase = jnp.full((L,), (f - lo) * VPAD, jnp.int32)
            for j in range(NCHUNK):
                for o in range(CHUNK // L):
                    sl = pl.ds(o * L, L)
                    idxv[f, j, sl] = idxv[f, j, sl] + fbase
            for j in range(NCHUNK):
                pltpu.async_copy(
                    table_hbm.at[idxv.at[f, j]],
                    valsv.at[f, pl.ds(j * CHUNK, CHUNK)],
                    sem,
                )
            return 0
        return off_fire_f

    lax.fori_loop(0, FH, make_off_fire(ta_hbm, 0), 0)
    lax.fori_loop(FH, F, make_off_fire(tb_hbm, FH), 0)

    def make_drain(table_hbm):
        def drain_f(f, _):
            for j in range(NCHUNK):
                pltpu.make_async_copy(
                    table_hbm.at[idxv.at[f, j]],
                    valsv.at[f, pl.ds(j * CHUNK, CHUNK)],
                    sem,
                ).wait()
            return 0
        return drain_f

    lax.fori_loop(0, FH, make_drain(ta_hbm), 0)
    lax.fori_loop(FH, F, make_drain(tb_hbm), 0)

    def red_c(c, _):
        acc = jnp.zeros((L,), jnp.float32)
        for f in range(F):
            acc = acc + valsv[f, pl.ds(c * L, L)]
        outv[pl.ds(c * L, L)] = acc
        return 0

    lax.fori_loop(0, BPW // L, red_c, 0)

    pltpu.sync_copy(outv, out_hbm.at[pl.ds(base, BPW)])


@jax.jit
def _linear_logit(xt, ta, tb):
    mesh = plsc.VectorSubcoreMesh(core_axis_name="c", subcore_axis_name="s")
    return pl.kernel(
        _body,
        out_type=jax.ShapeDtypeStruct((B,), jnp.float32),
        mesh=mesh,
        compiler_params=pltpu.CompilerParams(needs_layout_passes=False),
        scratch_types=[
            pltpu.VMEM((F, NCHUNK, CHUNK), jnp.int32),
            pltpu.VMEM((F, BPW), jnp.float32),
            pltpu.VMEM((BPW,), jnp.float32),
            pltpu.SemaphoreType.DMA,
        ],
    )(xt, ta, tb)


def kernel(X, tables):
    xt = X.astype(jnp.int32).T.reshape(F, NW, NCHUNK, CHUNK)
    t2 = tables.reshape(F, V)
    ta = jax.lax.optimization_barrier(t2[:FH]).reshape(FH * V)
    tb = jax.lax.optimization_barrier(t2[FH:]).reshape((F - FH) * V)
    return _linear_logit(xt, ta, tb)


# speedup vs baseline: 1.1330x; 1.1330x over previous
import jax
import jax.numpy as jnp
from jax import lax
from jax.experimental import pallas as pl
from jax.experimental.pallas import tpu as pltpu, tpu_sc as plsc

B = 16384
F = 26
V = 100000

NC = 2
NS = 16
L = 16
VPAD = 100000
NW = NC * NS
BPW = B // NW
CHUNK = 128
NCHUNK = BPW // CHUNK


def _body(xt_hbm, table_hbm, out_hbm, idxv, valsv, outv, sem):
    wid = lax.axis_index("s") * NC + lax.axis_index("c")
    base = wid * BPW

    pltpu.sync_copy(xt_hbm.at[:, wid], idxv)

    def off_fire_f(f, _):
        fbase = jnp.full((L,), f * VPAD, jnp.int32)
        for j in range(NCHUNK):
            for o in range(CHUNK // L):
                sl = pl.ds(o * L, L)
                idxv[f, j, sl] = idxv[f, j, sl] + fbase
        for j in range(NCHUNK):
            pltpu.async_copy(
                table_hbm.at[idxv.at[f, j]],
                valsv.at[f, pl.ds(j * CHUNK, CHUNK)],
                sem,
            )
        return 0

    lax.fori_loop(0, F, off_fire_f, 0)

    def drain_f(f, _):
        for j in range(NCHUNK):
            pltpu.make_async_copy(
                table_hbm.at[idxv.at[f, j]],
                valsv.at[f, pl.ds(j * CHUNK, CHUNK)],
                sem,
            ).wait()
        return 0

    lax.fori_loop(0, F, drain_f, 0)

    def red_c(c, _):
        acc = jnp.zeros((L,), jnp.float32)
        for f in range(F):
            acc = acc + valsv[f, pl.ds(c * L, L)]
        outv[pl.ds(c * L, L)] = acc
        return 0

    lax.fori_loop(0, BPW // L, red_c, 0)

    pltpu.sync_copy(outv, out_hbm.at[pl.ds(base, BPW)])


@jax.jit
def _linear_logit(xt, table_flat):
    mesh = plsc.VectorSubcoreMesh(core_axis_name="c", subcore_axis_name="s")
    return pl.kernel(
        _body,
        out_type=jax.ShapeDtypeStruct((B,), jnp.float32),
        mesh=mesh,
        compiler_params=pltpu.CompilerParams(needs_layout_passes=False),
        scratch_types=[
            pltpu.VMEM((F, NCHUNK, CHUNK), jnp.int32),
            pltpu.VMEM((F, BPW), jnp.float32),
            pltpu.VMEM((BPW,), jnp.float32),
            pltpu.SemaphoreType.DMA,
        ],
    )(xt, table_flat)


def kernel(X, tables):
    xt = X.astype(jnp.int32).T.reshape(F, NW, NCHUNK, CHUNK)
    t2 = jax.lax.optimization_barrier(tables.reshape(F, V))
    table_flat = t2.reshape(F * V)
    return _linear_logit(xt, table_flat)
